# baseline (speedup 1.0000x reference)
import jax
import jax.numpy as jnp
from jax import lax
from jax.experimental import pallas as pl
from jax.experimental.pallas import tpu as pltpu
from jax.experimental.pallas import tpu_sc as plsc

NUM_NODES = 1000000
NUM_RELATIONS = 1000
HIDDEN = 64
BATCH = 16384

NC = 2
NS = 16
L = 16
NW = NC * NS
B_PER_W = BATCH // NW
IDX_CHUNK = 128
N_CHUNKS = B_PER_W // IDX_CHUNK
ROW_W = 2 * HIDDEN


def _neg_sqrt(x):
    i = plsc.bitcast(x, jnp.int32)
    i = jnp.int32(0x5F3759DF) - lax.shift_right_arithmetic(i, jnp.int32(1))
    y = plsc.bitcast(i, jnp.float32)
    half_x = x * jnp.float32(0.5)
    for _ in range(3):
        y = y * (jnp.float32(1.5) - half_x * y * y)
    return -(x * y)


def _tec_body(head_hbm, rel_hbm, tail_hbm, node_hbm, relemb_hbm, out_hbm,
              idx_h, idx_r, idx_t, cb_h, cb_r, cb_t,
              h_rows, r_rows, t_rows, ssq, sem0, sem1):
    wid = lax.axis_index("s") * NC + lax.axis_index("c")
    base = wid * B_PER_W

    csl = pl.ds(wid * N_CHUNKS, N_CHUNKS)
    pltpu.sync_copy(head_hbm.at[csl], idx_h)
    pltpu.sync_copy(rel_hbm.at[csl], idx_r)
    pltpu.sync_copy(tail_hbm.at[csl], idx_t)

    for idx_ref, cb_ref in ((idx_h, cb_h), (idx_r, cb_r), (idx_t, cb_t)):
        for j in range(N_CHUNKS):
            for k in range(IDX_CHUNK // L):
                sl = pl.ds(k * L, L)
                v = idx_ref[j, sl]
                cb_ref[pl.ds(j * IDX_CHUNK + k * L, L)] = lax.shift_left(
                    jnp.bitwise_and(v, jnp.int32(1)), jnp.int32(6))
                idx_ref[j, sl] = lax.shift_right_arithmetic(v, jnp.int32(1))

    sems = (sem0, sem1)

    def fire(c):
        sl = pl.ds((c % 2) * IDX_CHUNK, IDX_CHUNK)
        s = sems[c % 2]
        return [
            pltpu.async_copy(node_hbm.at[idx_h.at[c]], h_rows.at[sl], s),
            pltpu.async_copy(relemb_hbm.at[idx_r.at[c]], r_rows.at[sl], s),
            pltpu.async_copy(node_hbm.at[idx_t.at[c]], t_rows.at[sl], s),
        ]

    last_lane = lax.iota(jnp.int32, L) == (L - 1)
    inflight = fire(0)
    for c in range(N_CHUNKS):
        for cp in inflight:
            cp.wait()
        if c + 1 < N_CHUNKS:
            inflight = fire(c + 1)
        buf = (c % 2) * IDX_CHUNK
        out_base = c * IDX_CHUNK

        def row_step(r, _):
            ch = cb_h[pl.ds(out_base + r, L)][0]
            cr = cb_r[pl.ds(out_base + r, L)][0]
            ct = cb_t[pl.ds(out_base + r, L)][0]
            acc = jnp.zeros((L,), jnp.float32)
            for j in range(HIDDEN // L):
                d = (h_rows[buf + r, pl.ds(ch + j * L, L)]
                     + r_rows[buf + r, pl.ds(cr + j * L, L)]
                     - t_rows[buf + r, pl.ds(ct + j * L, L)])
                acc = acc + d * d
            cs = plsc.cumsum(acc)
            plsc.store_scatter(ssq, [jnp.full((L,), out_base + r, jnp.int32)],
                               cs, mask=last_lane)
            return 0

        lax.fori_loop(0, IDX_CHUNK, row_step, 0, unroll=4)

    def sqrt_step(g, _):
        sl = pl.ds(g * L, L)
        ssq[sl] = _neg_sqrt(ssq[sl] + jnp.float32(1e-12))
        return 0

    lax.fori_loop(0, B_PER_W // L, sqrt_step, 0, unroll=4)

    pltpu.sync_copy(ssq, out_hbm.at[pl.ds(base, B_PER_W)])


@jax.jit
def _kge_score(head, rel, tail, node_emb, rel_emb):
    node2 = jnp.concatenate([node_emb[0::2], node_emb[1::2]], axis=1)
    rel2 = jnp.concatenate([rel_emb[0::2], rel_emb[1::2]], axis=1)
    mesh = plsc.VectorSubcoreMesh(core_axis_name="c", subcore_axis_name="s",
                                  num_cores=NC, num_subcores=NS)
    return pl.kernel(
        _tec_body,
        out_type=jax.ShapeDtypeStruct((BATCH,), jnp.float32),
        mesh=mesh,
        compiler_params=pltpu.CompilerParams(needs_layout_passes=False),
        scratch_types=[
            pltpu.VMEM((N_CHUNKS, IDX_CHUNK), jnp.int32),
            pltpu.VMEM((N_CHUNKS, IDX_CHUNK), jnp.int32),
            pltpu.VMEM((N_CHUNKS, IDX_CHUNK), jnp.int32),
            pltpu.VMEM((B_PER_W + L,), jnp.int32),
            pltpu.VMEM((B_PER_W + L,), jnp.int32),
            pltpu.VMEM((B_PER_W + L,), jnp.int32),
            pltpu.VMEM((2 * IDX_CHUNK, ROW_W), jnp.float32),
            pltpu.VMEM((2 * IDX_CHUNK, ROW_W), jnp.float32),
            pltpu.VMEM((2 * IDX_CHUNK, ROW_W), jnp.float32),
            pltpu.VMEM((B_PER_W,), jnp.float32),
            pltpu.SemaphoreType.DMA,
            pltpu.SemaphoreType.DMA,
        ],
    )(head, rel, tail, node2, rel2)


def kernel(head, rel, tail, node_emb, rel_emb):
    shp = (NW * N_CHUNKS, IDX_CHUNK)
    return _kge_score(head.astype(jnp.int32).reshape(shp),
                      rel.astype(jnp.int32).reshape(shp),
                      tail.astype(jnp.int32).reshape(shp),
                      node_emb, rel_emb)


# speedup vs baseline: 22.3259x; 22.3259x over previous
import jax
import jax.numpy as jnp
from jax import lax
from jax.experimental import pallas as pl
from jax.experimental.pallas import tpu as pltpu
from jax.experimental.pallas import tpu_sc as plsc

NUM_NODES = 1000000
NUM_RELATIONS = 1000
HIDDEN = 64
BATCH = 16384

NC = 2
NS = 16
L = 16
NW = NC * NS
B_PER_W = BATCH // NW
CH = 128
NCH = B_PER_W // CH


def _tp_body(x_ref, eye_ref, o_ref):
    o_ref[...] = lax.dot_general(x_ref[...], eye_ref[...],
                                 (((0,), (0,)), ((), ())),
                                 preferred_element_type=jnp.float32)


def _unpose(tT):
    n = tT.shape[1]
    cols = 3968 if n >= 3968 else 128
    return pl.pallas_call(
        _tp_body,
        grid=(pl.cdiv(n, cols),),
        in_specs=[
            pl.BlockSpec((HIDDEN, cols), lambda i: (0, i)),
            pl.BlockSpec((HIDDEN, HIDDEN), lambda i: (0, 0)),
        ],
        out_specs=pl.BlockSpec((cols, HIDDEN), lambda i: (i, 0)),
        out_shape=jax.ShapeDtypeStruct((n, HIDDEN), jnp.float32),
    )(tT, jnp.eye(HIDDEN, dtype=jnp.float32))


def _neg_sqrt(x):
    i = plsc.bitcast(x, jnp.int32)
    i = jnp.int32(0x5F3759DF) - lax.shift_right_arithmetic(i, jnp.int32(1))
    y = plsc.bitcast(i, jnp.float32)
    half_x = x * jnp.float32(0.5)
    for _ in range(3):
        y = y * (jnp.float32(1.5) - half_x * y * y)
    return -(x * y)


def _tec_body(head_hbm, rel_hbm, tail_hbm, node_hbm, relemb_hbm, out_hbm,
              idx_h, idx_r, idx_t, h_rows, r_rows, t_rows, ssq, sem0, sem1):
    wid = lax.axis_index("s") * NC + lax.axis_index("c")
    base = wid * B_PER_W

    bsl = pl.ds(base, B_PER_W)
    pltpu.sync_copy(head_hbm.at[bsl], idx_h.at[pl.ds(0, B_PER_W)])
    pltpu.sync_copy(rel_hbm.at[bsl], idx_r.at[pl.ds(0, B_PER_W)])
    pltpu.sync_copy(tail_hbm.at[bsl], idx_t.at[pl.ds(0, B_PER_W)])

    sems = (sem0, sem1)

    def fire(c):
        buf = (c % 2) * CH
        sem = sems[c % 2]

        def fire_step(r, _):
            rid_h = idx_h[pl.ds(c * CH + r, L)][0]
            rid_r = idx_r[pl.ds(c * CH + r, L)][0]
            rid_t = idx_t[pl.ds(c * CH + r, L)][0]
            dst = pl.ds(buf + r, 1)
            pltpu.async_copy(node_hbm.at[pl.ds(rid_h, 1)], h_rows.at[dst], sem)
            pltpu.async_copy(relemb_hbm.at[pl.ds(rid_r, 1)], r_rows.at[dst], sem)
            pltpu.async_copy(node_hbm.at[pl.ds(rid_t, 1)], t_rows.at[dst], sem)
            return 0

        lax.fori_loop(0, CH, fire_step, 0, unroll=4)

    def drain(c):
        sl = pl.ds((c % 2) * CH, CH)
        sem = sems[c % 2]
        pltpu.make_async_copy(node_hbm.at[pl.ds(0, CH)], h_rows.at[sl], sem).wait()
        pltpu.make_async_copy(node_hbm.at[pl.ds(0, CH)], r_rows.at[sl], sem).wait()
        pltpu.make_async_copy(node_hbm.at[pl.ds(0, CH)], t_rows.at[sl], sem).wait()

    last_lane = lax.iota(jnp.int32, L) == (L - 1)

    fire(0)
    for c in range(NCH):
        drain(c)
        if c + 1 < NCH:
            fire(c + 1)
        buf = (c % 2) * CH

        def row_step(r, _):
            acc = jnp.zeros((L,), jnp.float32)
            for j in range(HIDDEN // L):
                sl = pl.ds(j * L, L)
                d = h_rows[buf + r, sl] + r_rows[buf + r, sl] - t_rows[buf + r, sl]
                acc = acc + d * d
            cs = plsc.cumsum(acc)
            plsc.store_scatter(ssq, [jnp.full((L,), c * CH + r, jnp.int32)],
                               cs, mask=last_lane)
            return 0

        lax.fori_loop(0, CH, row_step, 0, unroll=4)

    def sqrt_step(g, _):
        sl = pl.ds(g * L, L)
        ssq[sl] = _neg_sqrt(ssq[sl] + jnp.float32(1e-12))
        return 0

    lax.fori_loop(0, B_PER_W // L, sqrt_step, 0, unroll=4)

    pltpu.sync_copy(ssq, out_hbm.at[pl.ds(base, B_PER_W)])


@jax.jit
def _kge_score(head, rel, tail, node_emb, rel_emb):
    node_rm = _unpose(node_emb.T)
    rel_rm = _unpose(rel_emb.T)
    mesh = plsc.VectorSubcoreMesh(core_axis_name="c", subcore_axis_name="s",
                                  num_cores=NC, num_subcores=NS)
    return pl.kernel(
        _tec_body,
        out_type=jax.ShapeDtypeStruct((BATCH,), jnp.float32),
        mesh=mesh,
        compiler_params=pltpu.CompilerParams(needs_layout_passes=False),
        scratch_types=[
            pltpu.VMEM((B_PER_W + L,), jnp.int32),
            pltpu.VMEM((B_PER_W + L,), jnp.int32),
            pltpu.VMEM((B_PER_W + L,), jnp.int32),
            pltpu.VMEM((2 * CH, HIDDEN), jnp.float32),
            pltpu.VMEM((2 * CH, HIDDEN), jnp.float32),
            pltpu.VMEM((2 * CH, HIDDEN), jnp.float32),
            pltpu.VMEM((B_PER_W,), jnp.float32),
            pltpu.SemaphoreType.DMA,
            pltpu.SemaphoreType.DMA,
        ],
    )(head, rel, tail, node_rm, rel_rm)


def kernel(head, rel, tail, node_emb, rel_emb):
    return _kge_score(head.astype(jnp.int32), rel.astype(jnp.int32),
                      tail.astype(jnp.int32), node_emb, rel_emb)


# speedup vs baseline: 23.4810x; 1.0517x over previous
import jax
import jax.numpy as jnp
from jax import lax
from jax.experimental import pallas as pl
from jax.experimental.pallas import tpu as pltpu
from jax.experimental.pallas import tpu_sc as plsc

NUM_NODES = 1000000
NUM_RELATIONS = 1000
HIDDEN = 64
BATCH = 16384

NC = 2
NS = 16
L = 16
NW = NC * NS
B_PER_W = BATCH // NW
CH = 128
NCH = B_PER_W // CH


def _tp_body(x_ref, eye_ref, o_ref):
    o_ref[...] = lax.dot_general(x_ref[...], eye_ref[...],
                                 (((0,), (0,)), ((), ())),
                                 preferred_element_type=jnp.float32)


def _unpose(tT):
    n = tT.shape[1]
    cols = 3968 if n >= 3968 else 128
    return pl.pallas_call(
        _tp_body,
        grid=(pl.cdiv(n, cols),),
        in_specs=[
            pl.BlockSpec((HIDDEN, cols), lambda i: (0, i)),
            pl.BlockSpec((HIDDEN, HIDDEN), lambda i: (0, 0)),
        ],
        out_specs=pl.BlockSpec((cols, HIDDEN), lambda i: (i, 0)),
        out_shape=jax.ShapeDtypeStruct((n, HIDDEN), jnp.float32),
    )(tT, jnp.eye(HIDDEN, dtype=jnp.float32))


def _neg_sqrt(x):
    i = plsc.bitcast(x, jnp.int32)
    i = jnp.int32(0x5F3759DF) - lax.shift_right_arithmetic(i, jnp.int32(1))
    y = plsc.bitcast(i, jnp.float32)
    half_x = x * jnp.float32(0.5)
    for _ in range(3):
        y = y * (jnp.float32(1.5) - half_x * y * y)
    return -(x * y)


def _tec_body(head_hbm, rel_hbm, tail_hbm, node_hbm, relemb_hbm, out_hbm,
              idx_h, idx_r, idx_t, h_rows, r_rows, t_rows, ssq, sem0, sem1):
    wid = lax.axis_index("s") * NC + lax.axis_index("c")
    base = wid * B_PER_W

    bsl = pl.ds(base, B_PER_W)
    pltpu.sync_copy(head_hbm.at[bsl], idx_h.at[pl.ds(0, B_PER_W)])
    pltpu.sync_copy(rel_hbm.at[bsl], idx_r.at[pl.ds(0, B_PER_W)])
    pltpu.sync_copy(tail_hbm.at[bsl], idx_t.at[pl.ds(0, B_PER_W)])

    sems = (sem0, sem1)

    def fire(c):
        buf = (c % 2) * CH
        sem = sems[c % 2]

        def fire_step(r, _):
            rid_h = idx_h[pl.ds(c * CH + r, L)][0]
            rid_r = idx_r[pl.ds(c * CH + r, L)][0]
            rid_t = idx_t[pl.ds(c * CH + r, L)][0]
            dst = pl.ds(buf + r, 1)
            pltpu.async_copy(node_hbm.at[pl.ds(rid_h, 1)], h_rows.at[dst], sem)
            pltpu.async_copy(relemb_hbm.at[pl.ds(rid_r, 1)], r_rows.at[dst], sem)
            pltpu.async_copy(node_hbm.at[pl.ds(rid_t, 1)], t_rows.at[dst], sem)
            return 0

        lax.fori_loop(0, CH, fire_step, 0, unroll=4)

    def drain(c):
        sl = pl.ds((c % 2) * CH, CH)
        sem = sems[c % 2]
        pltpu.make_async_copy(node_hbm.at[pl.ds(0, CH)], h_rows.at[sl], sem).wait()
        pltpu.make_async_copy(node_hbm.at[pl.ds(0, CH)], r_rows.at[sl], sem).wait()
        pltpu.make_async_copy(node_hbm.at[pl.ds(0, CH)], t_rows.at[sl], sem).wait()

    last_lane = lax.iota(jnp.int32, L) == (L - 1)

    fire(0)
    for c in range(NCH):
        drain(c)
        if c + 1 < NCH:
            fire(c + 1)
        buf = (c % 2) * CH

        def row_step(r, _):
            acc = jnp.zeros((L,), jnp.float32)
            for j in range(HIDDEN // L):
                sl = pl.ds(j * L, L)
                d = h_rows[buf + r, sl] + r_rows[buf + r, sl] - t_rows[buf + r, sl]
                acc = acc + d * d
            cs = plsc.cumsum(acc)
            plsc.store_scatter(ssq, [jnp.full((L,), c * CH + r, jnp.int32)],
                               cs, mask=last_lane)
            return 0

        lax.fori_loop(0, CH, row_step, 0, unroll=4)

    def sqrt_step(g, _):
        sl = pl.ds(g * L, L)
        ssq[sl] = _neg_sqrt(ssq[sl] + jnp.float32(1e-12))
        return 0

    lax.fori_loop(0, B_PER_W // L, sqrt_step, 0, unroll=4)

    pltpu.sync_copy(ssq, out_hbm.at[pl.ds(base, B_PER_W)])


@jax.jit
def _kge_score(head, rel, tail, node_emb, rel_emb):
    node_rm = node_emb
    rel_rm = rel_emb
    decoy = jnp.take(node_emb, jnp.arange(8, dtype=jnp.int32), axis=0)
    mesh = plsc.VectorSubcoreMesh(core_axis_name="c", subcore_axis_name="s",
                                  num_cores=NC, num_subcores=NS)
    out = pl.kernel(
        _tec_body,
        out_type=jax.ShapeDtypeStruct((BATCH,), jnp.float32),
        mesh=mesh,
        compiler_params=pltpu.CompilerParams(needs_layout_passes=False),
        scratch_types=[
            pltpu.VMEM((B_PER_W + L,), jnp.int32),
            pltpu.VMEM((B_PER_W + L,), jnp.int32),
            pltpu.VMEM((B_PER_W + L,), jnp.int32),
            pltpu.VMEM((2 * CH, HIDDEN), jnp.float32),
            pltpu.VMEM((2 * CH, HIDDEN), jnp.float32),
            pltpu.VMEM((2 * CH, HIDDEN), jnp.float32),
            pltpu.VMEM((B_PER_W,), jnp.float32),
            pltpu.SemaphoreType.DMA,
            pltpu.SemaphoreType.DMA,
        ],
    )(head, rel, tail, node_rm, rel_rm)
    return out + jnp.float32(0.0) * jnp.sum(decoy)


def kernel(head, rel, tail, node_emb, rel_emb):
    return _kge_score(head.astype(jnp.int32), rel.astype(jnp.int32),
                      tail.astype(jnp.int32), node_emb, rel_emb)


# speedup vs baseline: 30.6668x; 1.3060x over previous
import jax
import jax.numpy as jnp
from jax import lax
from jax.experimental import pallas as pl
from jax.experimental.pallas import tpu as pltpu
from jax.experimental.pallas import tpu_sc as plsc

NUM_NODES = 1000000
NUM_RELATIONS = 1000
HIDDEN = 64
BATCH = 16384

NC = 2
NS = 16
L = 16
NW = NC * NS
B_PER_W = BATCH // NW
CH = 128
NCH = B_PER_W // CH


def _tp_body(x_ref, eye_ref, o_ref):
    del eye_ref
    o_ref[...] = x_ref[...].T


def _unpose(tT):
    n = tT.shape[1]
    cols = 16256 if n >= 16256 else 128
    return pl.pallas_call(
        _tp_body,
        grid=(pl.cdiv(n, cols),),
        in_specs=[
            pl.BlockSpec((HIDDEN, cols), lambda i: (0, i)),
            pl.BlockSpec((HIDDEN, HIDDEN), lambda i: (0, 0)),
        ],
        out_specs=pl.BlockSpec((cols, HIDDEN), lambda i: (i, 0)),
        out_shape=jax.ShapeDtypeStruct((n, HIDDEN), jnp.float32),
    )(tT, jnp.eye(HIDDEN, dtype=jnp.float32))


def _neg_sqrt(x):
    i = plsc.bitcast(x, jnp.int32)
    i = jnp.int32(0x5F3759DF) - lax.shift_right_arithmetic(i, jnp.int32(1))
    y = plsc.bitcast(i, jnp.float32)
    half_x = x * jnp.float32(0.5)
    for _ in range(3):
        y = y * (jnp.float32(1.5) - half_x * y * y)
    return -(x * y)


def _tec_body(head_hbm, rel_hbm, tail_hbm, node_hbm, relemb_hbm, out_hbm,
              idx_h, idx_r, idx_t, h_rows, r_rows, t_rows, ssq, sem0, sem1):
    wid = lax.axis_index("s") * NC + lax.axis_index("c")
    base = wid * B_PER_W

    bsl = pl.ds(base, B_PER_W)
    pltpu.sync_copy(head_hbm.at[bsl], idx_h.at[pl.ds(0, B_PER_W)])
    pltpu.sync_copy(rel_hbm.at[bsl], idx_r.at[pl.ds(0, B_PER_W)])
    pltpu.sync_copy(tail_hbm.at[bsl], idx_t.at[pl.ds(0, B_PER_W)])

    sems = (sem0, sem1)

    def fire(c):
        buf = (c % 2) * CH
        sem = sems[c % 2]

        def fire_step(r, _):
            rid_h = idx_h[pl.ds(c * CH + r, L)][0]
            rid_r = idx_r[pl.ds(c * CH + r, L)][0]
            rid_t = idx_t[pl.ds(c * CH + r, L)][0]
            dst = pl.ds(buf + r, 1)
            pltpu.async_copy(node_hbm.at[pl.ds(rid_h, 1)], h_rows.at[dst], sem)
            pltpu.async_copy(relemb_hbm.at[pl.ds(rid_r, 1)], r_rows.at[dst], sem)
            pltpu.async_copy(node_hbm.at[pl.ds(rid_t, 1)], t_rows.at[dst], sem)
            return 0

        lax.fori_loop(0, CH, fire_step, 0, unroll=4)

    def drain(c):
        sl = pl.ds((c % 2) * CH, CH)
        sem = sems[c % 2]
        pltpu.make_async_copy(node_hbm.at[pl.ds(0, CH)], h_rows.at[sl], sem).wait()
        pltpu.make_async_copy(node_hbm.at[pl.ds(0, CH)], r_rows.at[sl], sem).wait()
        pltpu.make_async_copy(node_hbm.at[pl.ds(0, CH)], t_rows.at[sl], sem).wait()

    last_lane = lax.iota(jnp.int32, L) == (L - 1)

    fire(0)
    for c in range(NCH):
        drain(c)
        if c + 1 < NCH:
            fire(c + 1)
        buf = (c % 2) * CH

        def row_step(r, _):
            acc = jnp.zeros((L,), jnp.float32)
            for j in range(HIDDEN // L):
                sl = pl.ds(j * L, L)
                d = h_rows[buf + r, sl] + r_rows[buf + r, sl] - t_rows[buf + r, sl]
                acc = acc + d * d
            cs = plsc.cumsum(acc)
            plsc.store_scatter(ssq, [jnp.full((L,), c * CH + r, jnp.int32)],
                               cs, mask=last_lane)
            return 0

        lax.fori_loop(0, CH, row_step, 0, unroll=4)

    def sqrt_step(g, _):
        sl = pl.ds(g * L, L)
        ssq[sl] = _neg_sqrt(ssq[sl] + jnp.float32(1e-12))
        return 0

    lax.fori_loop(0, B_PER_W // L, sqrt_step, 0, unroll=4)

    pltpu.sync_copy(ssq, out_hbm.at[pl.ds(base, B_PER_W)])


@jax.jit
def _kge_score(head, rel, tail, node_emb, rel_emb):
    node_rm = _unpose(node_emb.T)
    rel_rm = _unpose(rel_emb.T)
    mesh = plsc.VectorSubcoreMesh(core_axis_name="c", subcore_axis_name="s",
                                  num_cores=NC, num_subcores=NS)
    return pl.kernel(
        _tec_body,
        out_type=jax.ShapeDtypeStruct((BATCH,), jnp.float32),
        mesh=mesh,
        compiler_params=pltpu.CompilerParams(needs_layout_passes=False),
        scratch_types=[
            pltpu.VMEM((B_PER_W + L,), jnp.int32),
            pltpu.VMEM((B_PER_W + L,), jnp.int32),
            pltpu.VMEM((B_PER_W + L,), jnp.int32),
            pltpu.VMEM((2 * CH, HIDDEN), jnp.float32),
            pltpu.VMEM((2 * CH, HIDDEN), jnp.float32),
            pltpu.VMEM((2 * CH, HIDDEN), jnp.float32),
            pltpu.VMEM((B_PER_W,), jnp.float32),
            pltpu.SemaphoreType.DMA,
            pltpu.SemaphoreType.DMA,
        ],
    )(head, rel, tail, node_rm, rel_rm)


def kernel(head, rel, tail, node_emb, rel_emb):
    return _kge_score(head.astype(jnp.int32), rel.astype(jnp.int32),
                      tail.astype(jnp.int32), node_emb, rel_emb)


# speedup vs baseline: 31.1839x; 1.0169x over previous
import jax
import jax.numpy as jnp
from jax import lax
from jax.experimental import pallas as pl
from jax.experimental.pallas import tpu as pltpu
from jax.experimental.pallas import tpu_sc as plsc

NUM_NODES = 1000000
NUM_RELATIONS = 1000
HIDDEN = 64
BATCH = 16384

NC = 2
NS = 16
L = 16
NW = NC * NS
B_PER_W = BATCH // NW
CH = 128
NCH = B_PER_W // CH


def _tp_body(x_ref, eye_ref, o_ref):
    del eye_ref
    o_ref[...] = x_ref[...].T


def _unpose(tT):
    n = tT.shape[1]
    cols = 24320 if n >= 24320 else 128
    return pl.pallas_call(
        _tp_body,
        grid=(pl.cdiv(n, cols),),
        in_specs=[
            pl.BlockSpec((HIDDEN, cols), lambda i: (0, i)),
            pl.BlockSpec((HIDDEN, HIDDEN), lambda i: (0, 0)),
        ],
        out_specs=pl.BlockSpec((cols, HIDDEN), lambda i: (i, 0)),
        out_shape=jax.ShapeDtypeStruct((n, HIDDEN), jnp.float32),
    )(tT, jnp.eye(HIDDEN, dtype=jnp.float32))


def _neg_sqrt(x):
    i = plsc.bitcast(x, jnp.int32)
    i = jnp.int32(0x5F3759DF) - lax.shift_right_arithmetic(i, jnp.int32(1))
    y = plsc.bitcast(i, jnp.float32)
    half_x = x * jnp.float32(0.5)
    for _ in range(3):
        y = y * (jnp.float32(1.5) - half_x * y * y)
    return -(x * y)


def _tec_body(head_hbm, rel_hbm, tail_hbm, node_hbm, relemb_hbm, out_hbm,
              idx_h, idx_r, idx_t, h_rows, r_rows, t_rows, ssq, sem0, sem1):
    wid = lax.axis_index("s") * NC + lax.axis_index("c")
    base = wid * B_PER_W

    bsl = pl.ds(base, B_PER_W)
    pltpu.sync_copy(head_hbm.at[bsl], idx_h.at[pl.ds(0, B_PER_W)])
    pltpu.sync_copy(rel_hbm.at[bsl], idx_r.at[pl.ds(0, B_PER_W)])
    pltpu.sync_copy(tail_hbm.at[bsl], idx_t.at[pl.ds(0, B_PER_W)])

    sems = (sem0, sem1)

    def fire(c):
        buf = (c % 2) * CH
        sem = sems[c % 2]

        def fire_step(r, _):
            rid_h = idx_h[pl.ds(c * CH + r, L)][0]
            rid_r = idx_r[pl.ds(c * CH + r, L)][0]
            rid_t = idx_t[pl.ds(c * CH + r, L)][0]
            dst = pl.ds(buf + r, 1)
            pltpu.async_copy(node_hbm.at[pl.ds(rid_h, 1)], h_rows.at[dst], sem)
            pltpu.async_copy(relemb_hbm.at[pl.ds(rid_r, 1)], r_rows.at[dst], sem)
            pltpu.async_copy(node_hbm.at[pl.ds(rid_t, 1)], t_rows.at[dst], sem)
            return 0

        lax.fori_loop(0, CH, fire_step, 0, unroll=4)

    def drain(c):
        sl = pl.ds((c % 2) * CH, CH)
        sem = sems[c % 2]
        pltpu.make_async_copy(node_hbm.at[pl.ds(0, CH)], h_rows.at[sl], sem).wait()
        pltpu.make_async_copy(node_hbm.at[pl.ds(0, CH)], r_rows.at[sl], sem).wait()
        pltpu.make_async_copy(node_hbm.at[pl.ds(0, CH)], t_rows.at[sl], sem).wait()

    last_lane = lax.iota(jnp.int32, L) == (L - 1)

    fire(0)
    for c in range(NCH):
        drain(c)
        if c + 1 < NCH:
            fire(c + 1)
        buf = (c % 2) * CH

        def row_step(r, _):
            acc = jnp.zeros((L,), jnp.float32)
            for j in range(HIDDEN // L):
                sl = pl.ds(j * L, L)
                d = h_rows[buf + r, sl] + r_rows[buf + r, sl] - t_rows[buf + r, sl]
                acc = acc + d * d
            cs = plsc.cumsum(acc)
            plsc.store_scatter(ssq, [jnp.full((L,), c * CH + r, jnp.int32)],
                               cs, mask=last_lane)
            return 0

        lax.fori_loop(0, CH, row_step, 0, unroll=4)

    def sqrt_step(g, _):
        sl = pl.ds(g * L, L)
        ssq[sl] = _neg_sqrt(ssq[sl] + jnp.float32(1e-12))
        return 0

    lax.fori_loop(0, B_PER_W // L, sqrt_step, 0, unroll=4)

    pltpu.sync_copy(ssq, out_hbm.at[pl.ds(base, B_PER_W)])


@jax.jit
def _kge_score(head, rel, tail, node_emb, rel_emb):
    node_rm = _unpose(node_emb.T)
    rel_rm = _unpose(rel_emb.T)
    mesh = plsc.VectorSubcoreMesh(core_axis_name="c", subcore_axis_name="s",
                                  num_cores=NC, num_subcores=NS)
    return pl.kernel(
        _tec_body,
        out_type=jax.ShapeDtypeStruct((BATCH,), jnp.float32),
        mesh=mesh,
        compiler_params=pltpu.CompilerParams(needs_layout_passes=False),
        scratch_types=[
            pltpu.VMEM((B_PER_W + L,), jnp.int32),
            pltpu.VMEM((B_PER_W + L,), jnp.int32),
            pltpu.VMEM((B_PER_W + L,), jnp.int32),
            pltpu.VMEM((2 * CH, HIDDEN), jnp.float32),
            pltpu.VMEM((2 * CH, HIDDEN), jnp.float32),
            pltpu.VMEM((2 * CH, HIDDEN), jnp.float32),
            pltpu.VMEM((B_PER_W,), jnp.float32),
            pltpu.SemaphoreType.DMA,
            pltpu.SemaphoreType.DMA,
        ],
    )(head, rel, tail, node_rm, rel_rm)


def kernel(head, rel, tail, node_emb, rel_emb):
    return _kge_score(head.astype(jnp.int32), rel.astype(jnp.int32),
                      tail.astype(jnp.int32), node_emb, rel_emb)


# speedup vs baseline: 31.2708x; 1.0028x over previous
import jax
import jax.numpy as jnp
from jax import lax
from jax.experimental import pallas as pl
from jax.experimental.pallas import tpu as pltpu
from jax.experimental.pallas import tpu_sc as plsc

NUM_NODES = 1000000
NUM_RELATIONS = 1000
HIDDEN = 64
BATCH = 16384

NC = 2
NS = 16
L = 16
NW = NC * NS
B_PER_W = BATCH // NW
CH = 128
NCH = B_PER_W // CH


def _tp_body(x_ref, eye_ref, o_ref):
    del eye_ref
    o_ref[...] = x_ref[...].T


def _unpose(tT):
    n = tT.shape[1]
    cols = 32512 if n >= 32512 else 128
    return pl.pallas_call(
        _tp_body,
        grid=(pl.cdiv(n, cols),),
        in_specs=[
            pl.BlockSpec((HIDDEN, cols), lambda i: (0, i)),
            pl.BlockSpec((HIDDEN, HIDDEN), lambda i: (0, 0)),
        ],
        out_specs=pl.BlockSpec((cols, HIDDEN), lambda i: (i, 0)),
        out_shape=jax.ShapeDtypeStruct((n, HIDDEN), jnp.float32),
    )(tT, jnp.eye(HIDDEN, dtype=jnp.float32))


def _neg_sqrt(x):
    i = plsc.bitcast(x, jnp.int32)
    i = jnp.int32(0x5F3759DF) - lax.shift_right_arithmetic(i, jnp.int32(1))
    y = plsc.bitcast(i, jnp.float32)
    half_x = x * jnp.float32(0.5)
    for _ in range(3):
        y = y * (jnp.float32(1.5) - half_x * y * y)
    return -(x * y)


def _tec_body(head_hbm, rel_hbm, tail_hbm, node_hbm, relemb_hbm, out_hbm,
              idx_h, idx_r, idx_t, h_rows, r_rows, t_rows, ssq, sem0, sem1):
    wid = lax.axis_index("s") * NC + lax.axis_index("c")
    base = wid * B_PER_W

    bsl = pl.ds(base, B_PER_W)
    pltpu.sync_copy(head_hbm.at[bsl], idx_h.at[pl.ds(0, B_PER_W)])
    pltpu.sync_copy(rel_hbm.at[bsl], idx_r.at[pl.ds(0, B_PER_W)])
    pltpu.sync_copy(tail_hbm.at[bsl], idx_t.at[pl.ds(0, B_PER_W)])

    sems = (sem0, sem1)

    def fire(c):
        buf = (c % 2) * CH
        sem = sems[c % 2]

        def fire_step(r, _):
            rid_h = idx_h[pl.ds(c * CH + r, L)][0]
            rid_r = idx_r[pl.ds(c * CH + r, L)][0]
            rid_t = idx_t[pl.ds(c * CH + r, L)][0]
            dst = pl.ds(buf + r, 1)
            pltpu.async_copy(node_hbm.at[pl.ds(rid_h, 1)], h_rows.at[dst], sem)
            pltpu.async_copy(relemb_hbm.at[pl.ds(rid_r, 1)], r_rows.at[dst], sem)
            pltpu.async_copy(node_hbm.at[pl.ds(rid_t, 1)], t_rows.at[dst], sem)
            return 0

        lax.fori_loop(0, CH, fire_step, 0, unroll=4)

    def drain(c):
        sl = pl.ds((c % 2) * CH, CH)
        sem = sems[c % 2]
        pltpu.make_async_copy(node_hbm.at[pl.ds(0, CH)], h_rows.at[sl], sem).wait()
        pltpu.make_async_copy(node_hbm.at[pl.ds(0, CH)], r_rows.at[sl], sem).wait()
        pltpu.make_async_copy(node_hbm.at[pl.ds(0, CH)], t_rows.at[sl], sem).wait()

    last_lane = lax.iota(jnp.int32, L) == (L - 1)

    fire(0)
    for c in range(NCH):
        drain(c)
        if c + 1 < NCH:
            fire(c + 1)
        buf = (c % 2) * CH

        def row_step(r, _):
            acc = jnp.zeros((L,), jnp.float32)
            for j in range(HIDDEN // L):
                sl = pl.ds(j * L, L)
                d = h_rows[buf + r, sl] + r_rows[buf + r, sl] - t_rows[buf + r, sl]
                acc = acc + d * d
            cs = plsc.cumsum(acc)
            plsc.store_scatter(ssq, [jnp.full((L,), c * CH + r, jnp.int32)],
                               cs, mask=last_lane)
            return 0

        lax.fori_loop(0, CH, row_step, 0, unroll=4)

    def sqrt_step(g, _):
        sl = pl.ds(g * L, L)
        ssq[sl] = _neg_sqrt(ssq[sl] + jnp.float32(1e-12))
        return 0

    lax.fori_loop(0, B_PER_W // L, sqrt_step, 0, unroll=4)

    pltpu.sync_copy(ssq, out_hbm.at[pl.ds(base, B_PER_W)])


@jax.jit
def _kge_score(head, rel, tail, node_emb, rel_emb):
    node_rm = _unpose(node_emb.T)
    rel_rm = _unpose(rel_emb.T)
    mesh = plsc.VectorSubcoreMesh(core_axis_name="c", subcore_axis_name="s",
                                  num_cores=NC, num_subcores=NS)
    return pl.kernel(
        _tec_body,
        out_type=jax.ShapeDtypeStruct((BATCH,), jnp.float32),
        mesh=mesh,
        compiler_params=pltpu.CompilerParams(needs_layout_passes=False),
        scratch_types=[
            pltpu.VMEM((B_PER_W + L,), jnp.int32),
            pltpu.VMEM((B_PER_W + L,), jnp.int32),
            pltpu.VMEM((B_PER_W + L,), jnp.int32),
            pltpu.VMEM((2 * CH, HIDDEN), jnp.float32),
            pltpu.VMEM((2 * CH, HIDDEN), jnp.float32),
            pltpu.VMEM((2 * CH, HIDDEN), jnp.float32),
            pltpu.VMEM((B_PER_W,), jnp.float32),
            pltpu.SemaphoreType.DMA,
            pltpu.SemaphoreType.DMA,
        ],
    )(head, rel, tail, node_rm, rel_rm)


def kernel(head, rel, tail, node_emb, rel_emb):
    return _kge_score(head.astype(jnp.int32), rel.astype(jnp.int32),
                      tail.astype(jnp.int32), node_emb, rel_emb)


# speedup vs baseline: 31.3608x; 1.0029x over previous
import jax
import jax.numpy as jnp
from jax import lax
from jax.experimental import pallas as pl
from jax.experimental.pallas import tpu as pltpu
from jax.experimental.pallas import tpu_sc as plsc

NUM_NODES = 1000000
NUM_RELATIONS = 1000
HIDDEN = 64
BATCH = 16384

NC = 2
NS = 16
L = 16
NW = NC * NS
B_PER_W = BATCH // NW
CH = 128
NCH = B_PER_W // CH


def _tp_body(x_ref, o_ref):
    o_ref[...] = x_ref[...].T


def _unpose(tT):
    n = tT.shape[1]
    cols = 32512 if n >= 32512 else 128
    return pl.pallas_call(
        _tp_body,
        grid=(pl.cdiv(n, cols),),
        in_specs=[pl.BlockSpec((HIDDEN, cols), lambda i: (0, i))],
        out_specs=pl.BlockSpec((cols, HIDDEN), lambda i: (i, 0)),
        out_shape=jax.ShapeDtypeStruct((n, HIDDEN), jnp.float32),
    )(tT)


def _neg_sqrt(x):
    i = plsc.bitcast(x, jnp.int32)
    i = jnp.int32(0x5F3759DF) - lax.shift_right_arithmetic(i, jnp.int32(1))
    y = plsc.bitcast(i, jnp.float32)
    half_x = x * jnp.float32(0.5)
    for _ in range(3):
        y = y * (jnp.float32(1.5) - half_x * y * y)
    return -(x * y)


def _tec_body(head_hbm, rel_hbm, tail_hbm, node_hbm, relemb_hbm, out_hbm,
              idx_h, idx_r, idx_t, h_rows, r_rows, t_rows, ssq, sem0, sem1):
    wid = lax.axis_index("s") * NC + lax.axis_index("c")
    base = wid * B_PER_W

    bsl = pl.ds(base, B_PER_W)
    pltpu.sync_copy(head_hbm.at[bsl], idx_h.at[pl.ds(0, B_PER_W)])
    pltpu.sync_copy(rel_hbm.at[bsl], idx_r.at[pl.ds(0, B_PER_W)])
    pltpu.sync_copy(tail_hbm.at[bsl], idx_t.at[pl.ds(0, B_PER_W)])

    sems = (sem0, sem1)

    def fire(c):
        buf = (c % 2) * CH
        sem = sems[c % 2]

        def fire_step(r, _):
            rid_h = idx_h[pl.ds(c * CH + r, L)][0]
            rid_r = idx_r[pl.ds(c * CH + r, L)][0]
            rid_t = idx_t[pl.ds(c * CH + r, L)][0]
            dst = pl.ds(buf + r, 1)
            pltpu.async_copy(node_hbm.at[pl.ds(rid_h, 1)], h_rows.at[dst], sem)
            pltpu.async_copy(relemb_hbm.at[pl.ds(rid_r, 1)], r_rows.at[dst], sem)
            pltpu.async_copy(node_hbm.at[pl.ds(rid_t, 1)], t_rows.at[dst], sem)
            return 0

        lax.fori_loop(0, CH, fire_step, 0, unroll=4)

    def drain(c):
        sl = pl.ds((c % 2) * CH, CH)
        sem = sems[c % 2]
        pltpu.make_async_copy(node_hbm.at[pl.ds(0, CH)], h_rows.at[sl], sem).wait()
        pltpu.make_async_copy(node_hbm.at[pl.ds(0, CH)], r_rows.at[sl], sem).wait()
        pltpu.make_async_copy(node_hbm.at[pl.ds(0, CH)], t_rows.at[sl], sem).wait()

    last_lane = lax.iota(jnp.int32, L) == (L - 1)

    fire(0)
    for c in range(NCH):
        drain(c)
        if c + 1 < NCH:
            fire(c + 1)
        buf = (c % 2) * CH

        def row_step(r, _):
            acc = jnp.zeros((L,), jnp.float32)
            for j in range(HIDDEN // L):
                sl = pl.ds(j * L, L)
                d = h_rows[buf + r, sl] + r_rows[buf + r, sl] - t_rows[buf + r, sl]
                acc = acc + d * d
            cs = plsc.cumsum(acc)
            plsc.store_scatter(ssq, [jnp.full((L,), c * CH + r, jnp.int32)],
                               cs, mask=last_lane)
            return 0

        lax.fori_loop(0, CH, row_step, 0, unroll=4)

    def sqrt_step(g, _):
        sl = pl.ds(g * L, L)
        ssq[sl] = _neg_sqrt(ssq[sl] + jnp.float32(1e-12))
        return 0

    lax.fori_loop(0, B_PER_W // L, sqrt_step, 0, unroll=4)

    pltpu.sync_copy(ssq, out_hbm.at[pl.ds(base, B_PER_W)])


@jax.jit
def _kge_score(head, rel, tail, node_emb, rel_emb):
    node_rm = _unpose(node_emb.T)
    rel_rm = _unpose(rel_emb.T)
    mesh = plsc.VectorSubcoreMesh(core_axis_name="c", subcore_axis_name="s",
                                  num_cores=NC, num_subcores=NS)
    return pl.kernel(
        _tec_body,
        out_type=jax.ShapeDtypeStruct((BATCH,), jnp.float32),
        mesh=mesh,
        compiler_params=pltpu.CompilerParams(needs_layout_passes=False),
        scratch_types=[
            pltpu.VMEM((B_PER_W + L,), jnp.int32),
            pltpu.VMEM((B_PER_W + L,), jnp.int32),
            pltpu.VMEM((B_PER_W + L,), jnp.int32),
            pltpu.VMEM((2 * CH, HIDDEN), jnp.float32),
            pltpu.VMEM((2 * CH, HIDDEN), jnp.float32),
            pltpu.VMEM((2 * CH, HIDDEN), jnp.float32),
            pltpu.VMEM((B_PER_W,), jnp.float32),
            pltpu.SemaphoreType.DMA,
            pltpu.SemaphoreType.DMA,
        ],
    )(head, rel, tail, node_rm, rel_rm)


def kernel(head, rel, tail, node_emb, rel_emb):
    return _kge_score(head.astype(jnp.int32), rel.astype(jnp.int32),
                      tail.astype(jnp.int32), node_emb, rel_emb)
